# baseline (speedup 1.0000x reference)
import functools

import jax
import jax.numpy as jnp
from jax.experimental import pallas as pl
from jax.experimental.pallas import tpu as pltpu

N = 4096
F = 128
LATENT = 32
KSEL = N // 2
RB = 1024
RBG = 2048


def _prep_body(a_ref, abf_ref, dis_ref):
    a = a_ref[...]
    abf_ref[...] = a.astype(jnp.bfloat16)
    d = jnp.sum(a, axis=1, keepdims=True)
    dis_ref[...] = jnp.where(d > 0, jax.lax.rsqrt(jnp.maximum(d, 1.0)), 0.0)


def _prep_call(A):
    return pl.pallas_call(
        _prep_body,
        grid=(N // RB,),
        in_specs=[pl.BlockSpec((RB, N), lambda g: (g, 0))],
        out_specs=[pl.BlockSpec((RB, N), lambda g: (g, 0)),
                   pl.BlockSpec((RB, 1), lambda g: (g, 0))],
        out_shape=[jax.ShapeDtypeStruct((N, N), jnp.bfloat16),
                   jax.ShapeDtypeStruct((N, 1), jnp.float32)],
    )(A)


def _act(o, epilogue):
    if epilogue == "relu":
        return jax.nn.relu(o)
    if epilogue == "softplus":
        return jax.nn.softplus(o)
    return o


def _gcn_body(a_ref, x_ref, w_ref, b_ref, dis_ref, out_ref, v_ref, *, epilogue):
    g = pl.program_id(0)

    @pl.when(g == 0)
    def _():
        v = jnp.dot(dis_ref[...] * x_ref[...], w_ref[...],
                    preferred_element_type=jnp.float32)
        v_ref[...] = v.astype(jnp.bfloat16)

    s = jnp.dot(a_ref[...], v_ref[...], preferred_element_type=jnp.float32)
    disb = dis_ref[pl.ds(g * RBG, RBG), :]
    out_ref[...] = _act(-disb * s + b_ref[...], epilogue)


def _gcn_y_body(a_ref, x_ref, w_ref, b_ref, dis_ref, p_ref, out_ref, y_ref,
                v_ref, *, epilogue):
    g = pl.program_id(0)
    cout = out_ref.shape[1]

    @pl.when(g == 0)
    def _():
        v = jnp.dot(dis_ref[...] * x_ref[...], w_ref[...],
                    preferred_element_type=jnp.float32)
        vh = v.astype(jnp.bfloat16)
        vl = (v - vh.astype(jnp.float32)).astype(jnp.bfloat16)
        v_ref[:, :cout] = vh
        v_ref[:, cout:] = vl

    a = a_ref[...]
    s = (jnp.dot(a, v_ref[:, :cout], preferred_element_type=jnp.float32)
         + jnp.dot(a, v_ref[:, cout:], preferred_element_type=jnp.float32))
    disb = dis_ref[pl.ds(g * RB, RB), :]
    o = _act(-disb * s + b_ref[...], epilogue)
    out_ref[...] = o
    y_ref[...] = jnp.dot(o, p_ref[...], preferred_element_type=jnp.float32)


def _gcn_call(Abf, X, Wb, brow, dis, epilogue, pproj=None):
    Cin = X.shape[1]
    Cout = Wb.shape[1]
    grid = (N // RB,)
    in_specs = [
        pl.BlockSpec((RB, N), lambda g: (g, 0)),
        pl.BlockSpec((N, Cin), lambda g: (0, 0)),
        pl.BlockSpec((Cin, Cout), lambda g: (0, 0)),
        pl.BlockSpec((1, Cout), lambda g: (0, 0)),
        pl.BlockSpec((N, 1), lambda g: (0, 0)),
    ]
    if pproj is None:
        scratch = [pltpu.VMEM((N, Cout), jnp.bfloat16)]
        in_specs[0] = pl.BlockSpec((RBG, N), lambda g: (g, 0))
        return pl.pallas_call(
            functools.partial(_gcn_body, epilogue=epilogue),
            grid=(N // RBG,),
            in_specs=in_specs,
            out_specs=pl.BlockSpec((RBG, Cout), lambda g: (g, 0)),
            out_shape=jax.ShapeDtypeStruct((N, Cout), jnp.float32),
            scratch_shapes=scratch,
        )(Abf, X, Wb, brow, dis)
    in_specs.append(pl.BlockSpec((Cout, 128), lambda g: (0, 0)))
    scratch = [pltpu.VMEM((N, 2 * Cout), jnp.bfloat16)]
    return pl.pallas_call(
        functools.partial(_gcn_y_body, epilogue=epilogue),
        grid=grid,
        in_specs=in_specs,
        out_specs=[pl.BlockSpec((RB, Cout), lambda g: (g, 0)),
                   pl.BlockSpec((RB, 128), lambda g: (g, 0))],
        out_shape=[jax.ShapeDtypeStruct((N, Cout), jnp.float32),
                   jax.ShapeDtypeStruct((N, 128), jnp.float32)],
        scratch_shapes=scratch,
    )(Abf, X, Wb, brow, dis, pproj)


def _pool_scatter_call(IDX, EPS):
    from jax.experimental.pallas import tpu_sc as plsc
    from jax import lax

    mesh = plsc.VectorSubcoreMesh(core_axis_name="c", subcore_axis_name="s")

    @functools.partial(
        pl.kernel, mesh=mesh,
        out_type=jax.ShapeDtypeStruct((2, N, 128), jnp.float32),
        scratch_types=[pltpu.VMEM((128,), jnp.int32),
                       pltpu.VMEM((128, 128), jnp.float32),
                       pltpu.VMEM((256, 128), jnp.float32),
                       pltpu.SemaphoreType.DMA],
    )
    def k(idx_hbm, eps_hbm, epsf_hbm, idx_v, eps_v, zero_v, sem):
        b = lax.axis_index("c")
        sub = lax.axis_index("s")

        def init_body(i, carry):
            zv = jnp.zeros((16,), jnp.float32)
            for j in range(8):
                zero_v[i, pl.ds(16 * j, 16)] = zv
            return carry

        lax.fori_loop(0, 256, init_body, 0)

        r0 = sub * 256
        pltpu.sync_copy(zero_v, epsf_hbm.at[b, pl.ds(r0, 256)])
        plsc.subcore_barrier()

        s0 = sub * 128
        pltpu.sync_copy(idx_hbm.at[b, pl.ds(s0, 128)], idx_v)
        pltpu.sync_copy(eps_hbm.at[b, pl.ds(s0, 128)], eps_v)
        pltpu.async_copy(eps_v, epsf_hbm.at[b].at[idx_v], sem).wait()

    return k(IDX, EPS)


def _blockdiag(W):
    ci, co = W.shape
    Z = jnp.zeros((ci, co), W.dtype)
    return jnp.concatenate([
        jnp.concatenate([W, Z], axis=1),
        jnp.concatenate([Z, W], axis=1),
    ], axis=0)


def kernel(x, adjacency, W_enc0, b_enc0, p_pool0, W_encz, b_encz,
           W_dec0, b_dec0, W_dec1, b_dec1, W_out, b_out):
    Abf, dis = _prep_call(adjacency)

    xs = jnp.concatenate([x[0], x[1]], axis=1)
    Wb1 = _blockdiag(W_enc0)
    b1 = jnp.concatenate([b_enc0, b_enc0])[None, :]
    p0 = p_pool0 / jnp.linalg.norm(p_pool0)
    pproj = jnp.zeros((128, 128), jnp.float32)
    pproj = pproj.at[:64, 0].set(p0).at[64:, 1].set(p0)
    H, Yp = _gcn_call(Abf, xs, Wb1, b1, dis, "relu", pproj=pproj)
    y = jnp.stack([Yp[:, 0], Yp[:, 1]], axis=0)

    _, idx = jax.lax.top_k(y, KSEL)
    idx = jnp.sort(idx, axis=1)

    eps_tab = jnp.stack(
        [jax.random.normal(jax.random.fold_in(jax.random.key(42), b),
                           (KSEL, LATENT), jnp.float32) for b in range(2)],
        axis=0)
    eps_tab = jnp.concatenate(
        [eps_tab, jnp.ones((2, KSEL, 1), jnp.float32),
         jnp.zeros((2, KSEL, 128 - LATENT - 1), jnp.float32)], axis=2)
    EPSF = _pool_scatter_call(idx, eps_tab)
    m = EPSF[:, :, LATENT]

    tscale = jnp.tanh(y) * m
    G = jnp.concatenate([H[:, :64] * tscale[0][:, None],
                         H[:, 64:] * tscale[1][:, None]], axis=1)
    epss = [EPSF[0, :, :LATENT], EPSF[1, :, :LATENT]]

    Wb2 = _blockdiag(W_encz)
    b2 = jnp.concatenate([b_encz, b_encz])[None, :]
    S2 = _gcn_call(Abf, G, Wb2, b2, dis, "none")

    zs = []
    for b in range(2):
        mean = S2[:, b * 64:b * 64 + 32]
        lv = S2[:, b * 64 + 32:b * 64 + 64]
        zs.append(jnp.where(m[b][:, None] > 0,
                            mean + jnp.exp(0.5 * lv) * epss[b], 0.0))
    Z = jnp.concatenate(zs, axis=1)

    Wb3 = _blockdiag(W_dec0)
    b3 = jnp.concatenate([b_dec0, b_dec0])[None, :]
    S3 = _gcn_call(Abf, Z, Wb3, b3, dis, "relu")
    mcols = jnp.concatenate([jnp.tile(m[0][:, None], (1, 64)),
                             jnp.tile(m[1][:, None], (1, 64))], axis=1)
    V3 = S3 * mcols

    Wb4 = _blockdiag(W_dec1)
    b4 = jnp.concatenate([b_dec1, b_dec1])[None, :]
    S4 = _gcn_call(Abf, V3, Wb4, b4, dis, "relu")

    Wb5 = _blockdiag(W_out)
    b5 = jnp.concatenate([b_out, b_out])[None, :]
    S5 = _gcn_call(Abf, S4, Wb5, b5, dis, "softplus")

    return jnp.stack([S5[:, :F], S5[:, F:]], axis=0)


# speedup vs baseline: 1.0240x; 1.0240x over previous
import functools

import jax
import jax.numpy as jnp
from jax.experimental import pallas as pl
from jax.experimental.pallas import tpu as pltpu

N = 4096
F = 128
LATENT = 32
KSEL = N // 2
RB = 1024
RBG = 1024


def _prep_body(a_ref, abf_ref, dis_ref):
    a = a_ref[...]
    abf_ref[...] = a.astype(jnp.bfloat16)
    d = jnp.sum(a, axis=1, keepdims=True)
    dis_ref[...] = jnp.where(d > 0, jax.lax.rsqrt(jnp.maximum(d, 1.0)), 0.0)


def _prep_call(A):
    return pl.pallas_call(
        _prep_body,
        grid=(N // RB,),
        in_specs=[pl.BlockSpec((RB, N), lambda g: (g, 0))],
        out_specs=[pl.BlockSpec((RB, N), lambda g: (g, 0)),
                   pl.BlockSpec((RB, 1), lambda g: (g, 0))],
        out_shape=[jax.ShapeDtypeStruct((N, N), jnp.bfloat16),
                   jax.ShapeDtypeStruct((N, 1), jnp.float32)],
    )(A)


def _act(o, epilogue):
    if epilogue == "relu":
        return jax.nn.relu(o)
    if epilogue == "softplus":
        return jax.nn.softplus(o)
    return o


def _gcn_body(a_ref, x_ref, w_ref, b_ref, dis_ref, out_ref, v_ref, *, epilogue):
    g = pl.program_id(0)

    @pl.when(g == 0)
    def _():
        v = jnp.dot(dis_ref[...] * x_ref[...], w_ref[...],
                    preferred_element_type=jnp.float32)
        v_ref[...] = v.astype(jnp.bfloat16)

    s = jnp.dot(a_ref[...], v_ref[...], preferred_element_type=jnp.float32)
    disb = dis_ref[pl.ds(g * RBG, RBG), :]
    out_ref[...] = _act(-disb * s + b_ref[...], epilogue)


def _gcn_y_body(a_ref, x_ref, w_ref, b_ref, dis_ref, p_ref, out_ref, y_ref,
                v_ref, *, epilogue):
    g = pl.program_id(0)
    cout = out_ref.shape[1]

    @pl.when(g == 0)
    def _():
        v = jnp.dot(dis_ref[...] * x_ref[...], w_ref[...],
                    preferred_element_type=jnp.float32)
        vh = v.astype(jnp.bfloat16)
        vl = (v - vh.astype(jnp.float32)).astype(jnp.bfloat16)
        v_ref[:, :cout] = vh
        v_ref[:, cout:] = vl

    a = a_ref[...]
    s = (jnp.dot(a, v_ref[:, :cout], preferred_element_type=jnp.float32)
         + jnp.dot(a, v_ref[:, cout:], preferred_element_type=jnp.float32))
    disb = dis_ref[pl.ds(g * RB, RB), :]
    o = _act(-disb * s + b_ref[...], epilogue)
    out_ref[...] = o
    y_ref[...] = jnp.dot(o, p_ref[...], preferred_element_type=jnp.float32)


def _gcn_call(Abf, X, Wb, brow, dis, epilogue, pproj=None):
    Cin = X.shape[1]
    Cout = Wb.shape[1]
    grid = (N // RB,)
    in_specs = [
        pl.BlockSpec((RB, N), lambda g: (g, 0)),
        pl.BlockSpec((N, Cin), lambda g: (0, 0)),
        pl.BlockSpec((Cin, Cout), lambda g: (0, 0)),
        pl.BlockSpec((1, Cout), lambda g: (0, 0)),
        pl.BlockSpec((N, 1), lambda g: (0, 0)),
    ]
    if pproj is None:
        scratch = [pltpu.VMEM((N, Cout), jnp.bfloat16)]
        in_specs[0] = pl.BlockSpec((RBG, N), lambda g: (g, 0))
        return pl.pallas_call(
            functools.partial(_gcn_body, epilogue=epilogue),
            grid=(N // RBG,),
            in_specs=in_specs,
            out_specs=pl.BlockSpec((RBG, Cout), lambda g: (g, 0)),
            out_shape=jax.ShapeDtypeStruct((N, Cout), jnp.float32),
            scratch_shapes=scratch,
        )(Abf, X, Wb, brow, dis)
    in_specs.append(pl.BlockSpec((Cout, 128), lambda g: (0, 0)))
    scratch = [pltpu.VMEM((N, 2 * Cout), jnp.bfloat16)]
    return pl.pallas_call(
        functools.partial(_gcn_y_body, epilogue=epilogue),
        grid=grid,
        in_specs=in_specs,
        out_specs=[pl.BlockSpec((RB, Cout), lambda g: (g, 0)),
                   pl.BlockSpec((RB, 128), lambda g: (g, 0))],
        out_shape=[jax.ShapeDtypeStruct((N, Cout), jnp.float32),
                   jax.ShapeDtypeStruct((N, 128), jnp.float32)],
        scratch_shapes=scratch,
    )(Abf, X, Wb, brow, dis, pproj)


def _pool_scatter_call(IDX, EPS):
    from jax.experimental.pallas import tpu_sc as plsc
    from jax import lax

    mesh = plsc.VectorSubcoreMesh(core_axis_name="c", subcore_axis_name="s")

    @functools.partial(
        pl.kernel, mesh=mesh,
        out_type=jax.ShapeDtypeStruct((2, N, 128), jnp.float32),
        scratch_types=[pltpu.VMEM((128,), jnp.int32),
                       pltpu.VMEM((128, 128), jnp.float32),
                       pltpu.VMEM((256, 128), jnp.float32),
                       pltpu.SemaphoreType.DMA],
    )
    def k(idx_hbm, eps_hbm, epsf_hbm, idx_v, eps_v, zero_v, sem):
        b = lax.axis_index("c")
        sub = lax.axis_index("s")

        def init_body(i, carry):
            zv = jnp.zeros((16,), jnp.float32)
            for j in range(8):
                zero_v[i, pl.ds(16 * j, 16)] = zv
            return carry

        lax.fori_loop(0, 256, init_body, 0)

        r0 = sub * 256
        pltpu.sync_copy(zero_v, epsf_hbm.at[b, pl.ds(r0, 256)])
        plsc.subcore_barrier()

        s0 = sub * 128
        pltpu.sync_copy(idx_hbm.at[b, pl.ds(s0, 128)], idx_v)
        pltpu.sync_copy(eps_hbm.at[b, pl.ds(s0, 128)], eps_v)
        pltpu.async_copy(eps_v, epsf_hbm.at[b].at[idx_v], sem).wait()

    return k(IDX, EPS)


def _blockdiag(W):
    ci, co = W.shape
    Z = jnp.zeros((ci, co), W.dtype)
    return jnp.concatenate([
        jnp.concatenate([W, Z], axis=1),
        jnp.concatenate([Z, W], axis=1),
    ], axis=0)


def kernel(x, adjacency, W_enc0, b_enc0, p_pool0, W_encz, b_encz,
           W_dec0, b_dec0, W_dec1, b_dec1, W_out, b_out):
    Abf, dis = _prep_call(adjacency)

    xs = jnp.concatenate([x[0], x[1]], axis=1)
    Wb1 = _blockdiag(W_enc0)
    b1 = jnp.concatenate([b_enc0, b_enc0])[None, :]
    p0 = p_pool0 / jnp.linalg.norm(p_pool0)
    pproj = jnp.zeros((128, 128), jnp.float32)
    pproj = pproj.at[:64, 0].set(p0).at[64:, 1].set(p0)
    H, Yp = _gcn_call(Abf, xs, Wb1, b1, dis, "relu", pproj=pproj)
    y = jnp.stack([Yp[:, 0], Yp[:, 1]], axis=0)

    _, idx = jax.lax.top_k(y, KSEL)
    idx = jnp.sort(idx, axis=1)

    eps_tab = jnp.stack(
        [jax.random.normal(jax.random.fold_in(jax.random.key(42), b),
                           (KSEL, LATENT), jnp.float32) for b in range(2)],
        axis=0)
    eps_tab = jnp.concatenate(
        [eps_tab, jnp.ones((2, KSEL, 1), jnp.float32),
         jnp.zeros((2, KSEL, 128 - LATENT - 1), jnp.float32)], axis=2)
    EPSF = _pool_scatter_call(idx, eps_tab)
    m = EPSF[:, :, LATENT]

    tscale = jnp.tanh(y) * m
    G = jnp.concatenate([H[:, :64] * tscale[0][:, None],
                         H[:, 64:] * tscale[1][:, None]], axis=1)
    epss = [EPSF[0, :, :LATENT], EPSF[1, :, :LATENT]]

    Wb2 = _blockdiag(W_encz)
    b2 = jnp.concatenate([b_encz, b_encz])[None, :]
    S2 = _gcn_call(Abf, G, Wb2, b2, dis, "none")

    zs = []
    for b in range(2):
        mean = S2[:, b * 64:b * 64 + 32]
        lv = S2[:, b * 64 + 32:b * 64 + 64]
        zs.append(jnp.where(m[b][:, None] > 0,
                            mean + jnp.exp(0.5 * lv) * epss[b], 0.0))
    Z = jnp.concatenate(zs, axis=1)

    Wb3 = _blockdiag(W_dec0)
    b3 = jnp.concatenate([b_dec0, b_dec0])[None, :]
    S3 = _gcn_call(Abf, Z, Wb3, b3, dis, "relu")
    mcols = jnp.concatenate([jnp.tile(m[0][:, None], (1, 64)),
                             jnp.tile(m[1][:, None], (1, 64))], axis=1)
    V3 = S3 * mcols

    Wb4 = _blockdiag(W_dec1)
    b4 = jnp.concatenate([b_dec1, b_dec1])[None, :]
    S4 = _gcn_call(Abf, V3, Wb4, b4, dis, "relu")

    Wb5 = _blockdiag(W_out)
    b5 = jnp.concatenate([b_out, b_out])[None, :]
    S5 = _gcn_call(Abf, S4, Wb5, b5, dis, "softplus")

    return jnp.stack([S5[:, :F], S5[:, F:]], axis=0)


# speedup vs baseline: 1.0801x; 1.0548x over previous
import functools

import jax
import jax.numpy as jnp
from jax.experimental import pallas as pl
from jax.experimental.pallas import tpu as pltpu

N = 4096
F = 128
LATENT = 32
KSEL = N // 2

import numpy as _np

def _eps_tables():
    tabs = [_np.asarray(jax.random.normal(
        jax.random.fold_in(jax.random.key(42), b), (KSEL, LATENT),
        jnp.float32)) for b in range(2)]
    tab = _np.stack(tabs, axis=0)
    pad = _np.zeros((2, KSEL, 128 - LATENT), _np.float32)
    pad[:, :, 0] = 1.0
    return _np.concatenate([tab, pad], axis=2)

_EPS_TAB = _eps_tables()
RB = 512
RBG = 1024


def _prep_body(a_ref, abf_ref, dis_ref):
    a = a_ref[...]
    abf_ref[...] = a.astype(jnp.bfloat16)
    d = jnp.sum(a, axis=1, keepdims=True)
    dis_ref[...] = jnp.where(d > 0, jax.lax.rsqrt(jnp.maximum(d, 1.0)), 0.0)


def _prep_call(A):
    return pl.pallas_call(
        _prep_body,
        grid=(N // RB,),
        in_specs=[pl.BlockSpec((RB, N), lambda g: (g, 0))],
        out_specs=[pl.BlockSpec((RB, N), lambda g: (g, 0)),
                   pl.BlockSpec((RB, 1), lambda g: (g, 0))],
        out_shape=[jax.ShapeDtypeStruct((N, N), jnp.bfloat16),
                   jax.ShapeDtypeStruct((N, 1), jnp.float32)],
    )(A)


def _act(o, epilogue):
    if epilogue == "relu":
        return jax.nn.relu(o)
    if epilogue == "softplus":
        return jax.nn.softplus(o)
    return o


def _gcn_body(a_ref, x_ref, w_ref, b_ref, dis_ref, out_ref, v_ref, *, epilogue):
    g = pl.program_id(0)

    @pl.when(g == 0)
    def _():
        v = jnp.dot(dis_ref[...] * x_ref[...], w_ref[...],
                    preferred_element_type=jnp.float32)
        v_ref[...] = v.astype(jnp.bfloat16)

    s = jnp.dot(a_ref[...], v_ref[...], preferred_element_type=jnp.float32)
    disb = dis_ref[pl.ds(g * RBG, RBG), :]
    out_ref[...] = _act(-disb * s + b_ref[...], epilogue)


def _gcn_y_body(a_ref, x_ref, w_ref, b_ref, dis_ref, p_ref, out_ref, y_ref,
                v_ref, *, epilogue):
    g = pl.program_id(0)
    cout = out_ref.shape[1]

    @pl.when(g == 0)
    def _():
        v = jnp.dot(dis_ref[...] * x_ref[...], w_ref[...],
                    preferred_element_type=jnp.float32)
        vh = v.astype(jnp.bfloat16)
        vl = (v - vh.astype(jnp.float32)).astype(jnp.bfloat16)
        v_ref[:, :cout] = vh
        v_ref[:, cout:] = vl

    a = a_ref[...]
    s = (jnp.dot(a, v_ref[:, :cout], preferred_element_type=jnp.float32)
         + jnp.dot(a, v_ref[:, cout:], preferred_element_type=jnp.float32))
    disb = dis_ref[pl.ds(g * RB, RB), :]
    o = _act(-disb * s + b_ref[...], epilogue)
    out_ref[...] = o
    y_ref[...] = jnp.dot(o, p_ref[...], preferred_element_type=jnp.float32)


def _gcn_call(Abf, X, Wb, brow, dis, epilogue, pproj=None):
    Cin = X.shape[1]
    Cout = Wb.shape[1]
    grid = (N // RB,)
    in_specs = [
        pl.BlockSpec((RB, N), lambda g: (g, 0)),
        pl.BlockSpec((N, Cin), lambda g: (0, 0)),
        pl.BlockSpec((Cin, Cout), lambda g: (0, 0)),
        pl.BlockSpec((1, Cout), lambda g: (0, 0)),
        pl.BlockSpec((N, 1), lambda g: (0, 0)),
    ]
    if pproj is None:
        scratch = [pltpu.VMEM((N, Cout), jnp.bfloat16)]
        in_specs[0] = pl.BlockSpec((RBG, N), lambda g: (g, 0))
        return pl.pallas_call(
            functools.partial(_gcn_body, epilogue=epilogue),
            grid=(N // RBG,),
            in_specs=in_specs,
            out_specs=pl.BlockSpec((RBG, Cout), lambda g: (g, 0)),
            out_shape=jax.ShapeDtypeStruct((N, Cout), jnp.float32),
            scratch_shapes=scratch,
        )(Abf, X, Wb, brow, dis)
    in_specs.append(pl.BlockSpec((Cout, 128), lambda g: (0, 0)))
    scratch = [pltpu.VMEM((N, 2 * Cout), jnp.bfloat16)]
    return pl.pallas_call(
        functools.partial(_gcn_y_body, epilogue=epilogue),
        grid=grid,
        in_specs=in_specs,
        out_specs=[pl.BlockSpec((RB, Cout), lambda g: (g, 0)),
                   pl.BlockSpec((RB, 128), lambda g: (g, 0))],
        out_shape=[jax.ShapeDtypeStruct((N, Cout), jnp.float32),
                   jax.ShapeDtypeStruct((N, 128), jnp.float32)],
        scratch_shapes=scratch,
    )(Abf, X, Wb, brow, dis, pproj)


def _pool_scatter_call(IDX, EPS):
    from jax.experimental.pallas import tpu_sc as plsc
    from jax import lax

    mesh = plsc.VectorSubcoreMesh(core_axis_name="c", subcore_axis_name="s")

    @functools.partial(
        pl.kernel, mesh=mesh,
        out_type=jax.ShapeDtypeStruct((2, N, 128), jnp.float32),
        scratch_types=[pltpu.VMEM((128,), jnp.int32),
                       pltpu.VMEM((128, 128), jnp.float32),
                       pltpu.VMEM((256, 128), jnp.float32),
                       pltpu.SemaphoreType.DMA],
    )
    def k(idx_hbm, eps_hbm, epsf_hbm, idx_v, eps_v, zero_v, sem):
        b = lax.axis_index("c")
        sub = lax.axis_index("s")

        def init_body(i, carry):
            zv = jnp.zeros((16,), jnp.float32)
            for j in range(8):
                zero_v[i, pl.ds(16 * j, 16)] = zv
            return carry

        lax.fori_loop(0, 256, init_body, 0)

        r0 = sub * 256
        pltpu.sync_copy(zero_v, epsf_hbm.at[b, pl.ds(r0, 256)])
        plsc.subcore_barrier()

        s0 = sub * 128
        pltpu.sync_copy(idx_hbm.at[b, pl.ds(s0, 128)], idx_v)
        pltpu.sync_copy(eps_hbm.at[b, pl.ds(s0, 128)], eps_v)
        pltpu.async_copy(eps_v, epsf_hbm.at[b].at[idx_v], sem).wait()

    return k(IDX, EPS)


def _blockdiag(W):
    ci, co = W.shape
    Z = jnp.zeros((ci, co), W.dtype)
    return jnp.concatenate([
        jnp.concatenate([W, Z], axis=1),
        jnp.concatenate([Z, W], axis=1),
    ], axis=0)


def kernel(x, adjacency, W_enc0, b_enc0, p_pool0, W_encz, b_encz,
           W_dec0, b_dec0, W_dec1, b_dec1, W_out, b_out):
    Abf, dis = _prep_call(adjacency)

    xs = jnp.concatenate([x[0], x[1]], axis=1)
    Wb1 = _blockdiag(W_enc0)
    b1 = jnp.concatenate([b_enc0, b_enc0])[None, :]
    p0 = p_pool0 / jnp.linalg.norm(p_pool0)
    pproj = jnp.zeros((128, 128), jnp.float32)
    pproj = pproj.at[:64, 0].set(p0).at[64:, 1].set(p0)
    H, Yp = _gcn_call(Abf, xs, Wb1, b1, dis, "relu", pproj=pproj)
    y = jnp.stack([Yp[:, 0], Yp[:, 1]], axis=0)

    _, idx = jax.lax.top_k(y, KSEL)
    idx = jnp.sort(idx, axis=1)

    EPSF = _pool_scatter_call(idx, jnp.asarray(_EPS_TAB))
    m = EPSF[:, :, LATENT]

    tscale = jnp.tanh(y) * m
    G = jnp.concatenate([H[:, :64] * tscale[0][:, None],
                         H[:, 64:] * tscale[1][:, None]], axis=1)
    epss = [EPSF[0, :, :LATENT], EPSF[1, :, :LATENT]]

    Wb2 = _blockdiag(W_encz)
    b2 = jnp.concatenate([b_encz, b_encz])[None, :]
    S2 = _gcn_call(Abf, G, Wb2, b2, dis, "none")

    zs = []
    for b in range(2):
        mean = S2[:, b * 64:b * 64 + 32]
        lv = S2[:, b * 64 + 32:b * 64 + 64]
        zs.append(jnp.where(m[b][:, None] > 0,
                            mean + jnp.exp(0.5 * lv) * epss[b], 0.0))
    Z = jnp.concatenate(zs, axis=1)

    Wb3 = _blockdiag(W_dec0)
    b3 = jnp.concatenate([b_dec0, b_dec0])[None, :]
    S3 = _gcn_call(Abf, Z, Wb3, b3, dis, "relu")
    mcols = jnp.concatenate([jnp.tile(m[0][:, None], (1, 64)),
                             jnp.tile(m[1][:, None], (1, 64))], axis=1)
    V3 = S3 * mcols

    Wb4 = _blockdiag(W_dec1)
    b4 = jnp.concatenate([b_dec1, b_dec1])[None, :]
    S4 = _gcn_call(Abf, V3, Wb4, b4, dis, "relu")

    Wb5 = _blockdiag(W_out)
    b5 = jnp.concatenate([b_out, b_out])[None, :]
    S5 = _gcn_call(Abf, S4, Wb5, b5, dis, "softplus")

    return jnp.stack([S5[:, :F], S5[:, F:]], axis=0)
